# baseline (speedup 1.0000x reference)
import functools

import jax
import jax.numpy as jnp
import numpy as np
from jax.experimental import pallas as pl

_BINS = [0.0, float(np.log1p(5.0)), float(np.log1p(25.0)),
         float(np.log1p(50.0)), float(np.log1p(100.0))]

_C = 512
_RB = 64


def _body(yp_ref, yt_ref, out_ref):
    i = pl.program_id(0)
    accs = [jnp.zeros((8, _C), jnp.float32) for _ in range(8)]
    for j in range(_RB // 8):
        t = yt_ref[pl.ds(j * 8, 8), :]
        p = yp_ref[pl.ds(j * 8, 8), :]
        d = p - t
        d2 = d * d
        ge = [jnp.where(t >= e, 1.0, 0.0) for e in _BINS]
        for b in range(4):
            ind = ge[b] - ge[b + 1]
            accs[b] = accs[b] + ind * d2
            accs[4 + b] = accs[4 + b] + ind

    @pl.when(i == 0)
    def _init():
        out_ref[...] = jnp.zeros_like(out_ref)

    out_ref[...] += jnp.stack(accs, axis=0)


@functools.partial(jax.jit, static_argnames=("interpret",))
def kernel(y_pred, y_true, interpret=False):
    n = y_pred.size
    rows = n // _C
    yp = y_pred.reshape(rows, _C)
    yt = y_true.reshape(rows, _C)
    acc = pl.pallas_call(
        _body,
        grid=(rows // _RB,),
        in_specs=[pl.BlockSpec((_RB, _C), lambda i: (i, 0))] * 2,
        out_specs=pl.BlockSpec((8, 8, _C), lambda i: (0, 0, 0)),
        out_shape=jax.ShapeDtypeStruct((8, 8, _C), jnp.float32),
        interpret=interpret,
    )(yp, yt)
    q = jnp.sum(acc[0:4], axis=(1, 2))
    s = jnp.sum(acc[4:8], axis=(1, 2))
    mse = q / jnp.maximum(s, 1.0)
    valid = s > 0
    mses = jnp.where(valid, mse, 0.0)
    w = jnp.where(valid, 1.0 / jnp.maximum(s, 1.0), 0.0)
    w = w / jnp.sum(w)
    return jnp.sqrt(jnp.sum(w * mses) + 1e-8)


# speedup vs baseline: 2.5065x; 2.5065x over previous
import functools

import jax
import jax.numpy as jnp
import numpy as np
from jax import lax
from jax.experimental import pallas as pl
from jax.experimental.pallas import tpu as pltpu
from jax.experimental.pallas import tpu_sc as plsc

_BINS = [0.0, float(np.log1p(5.0)), float(np.log1p(25.0)),
         float(np.log1p(50.0)), float(np.log1p(100.0))]

_C = 384
_ROWS = 36864
_NW = 32
_RPW = _ROWS // _NW
_CH = 64
_NCHUNK = _RPW // _CH
_VPR = _C // 16


def _sc_body(yp_hbm, yt_hbm, out_hbm, ypb0, ypb1, ytb0, ytb1,
             accq, accc, s0, s1, s2, s3):
    w = lax.axis_index("s") * 2 + lax.axis_index("c")
    base = w * _RPW
    ypb = [ypb0, ypb1]
    ytb = [ytb0, ytb1]
    sp = [s0, s1]
    st = [s2, s3]

    lanes6 = lax.iota(jnp.int32, 16) * 6
    zero16 = jnp.zeros((16,), jnp.float32)
    one16 = jnp.ones((16,), jnp.float32)
    for k in range(6):
        accq[pl.ds(k * 16, 16)] = zero16
        accc[pl.ds(k * 16, 16)] = zero16

    def issue(c, slot):
        r0 = base + c * _CH
        pltpu.async_copy(yp_hbm.at[pl.ds(r0, _CH)], ypb[slot], sp[slot])
        pltpu.async_copy(yt_hbm.at[pl.ds(r0, _CH)], ytb[slot], st[slot])

    def wait(c, slot):
        r0 = base + c * _CH
        pltpu.make_async_copy(yp_hbm.at[pl.ds(r0, _CH)], ypb[slot], sp[slot]).wait()
        pltpu.make_async_copy(yt_hbm.at[pl.ds(r0, _CH)], ytb[slot], st[slot]).wait()

    issue(0, 0)
    issue(1, 1)

    def chunk_pair(g, carry):
        for slot in range(2):
            c = g * 2 + slot
            wait(c, slot)

            @plsc.parallel_loop(0, _CH, 1, unroll=2)
            def _rows(r):
                for v in range(_VPR):
                    t = ytb[slot][r, pl.ds(v * 16, 16)]
                    p = ypb[slot][r, pl.ds(v * 16, 16)]
                    d = p - t
                    d2 = d * d
                    idx = jnp.zeros((16,), jnp.int32)
                    ione = jnp.ones((16,), jnp.int32)
                    izero = jnp.zeros((16,), jnp.int32)
                    for e in _BINS:
                        idx = idx + jnp.where(t >= e, ione, izero)
                    fidx = lanes6 + idx
                    plsc.addupdate_scatter(accq, [fidx], d2)
                    plsc.addupdate_scatter(accc, [fidx], one16)

            @pl.when(c + 2 < _NCHUNK)
            def _():
                issue(c + 2, slot)
        return carry

    lax.fori_loop(0, _NCHUNK // 2, chunk_pair, 0)

    pltpu.sync_copy(accq, out_hbm.at[w, 0])
    pltpu.sync_copy(accc, out_hbm.at[w, 1])


def sc_partials(yp, yt):
    mesh = plsc.VectorSubcoreMesh(core_axis_name="c", subcore_axis_name="s",
                                  num_cores=2, num_subcores=16)
    f = functools.partial(
        pl.kernel,
        mesh=mesh,
        compiler_params=pltpu.CompilerParams(needs_layout_passes=False),
        out_type=jax.ShapeDtypeStruct((_NW, 2, 96), jnp.float32),
        scratch_types=[
            pltpu.VMEM((_CH, _C), jnp.float32),
            pltpu.VMEM((_CH, _C), jnp.float32),
            pltpu.VMEM((_CH, _C), jnp.float32),
            pltpu.VMEM((_CH, _C), jnp.float32),
            pltpu.VMEM((96,), jnp.float32),
            pltpu.VMEM((96,), jnp.float32),
            pltpu.SemaphoreType.DMA,
            pltpu.SemaphoreType.DMA,
            pltpu.SemaphoreType.DMA,
            pltpu.SemaphoreType.DMA,
        ],
    )(_sc_body)
    return f(yp, yt)


def kernel(y_pred, y_true):
    yp = y_pred.reshape(_ROWS, _C)
    yt = y_true.reshape(_ROWS, _C)
    acc = sc_partials(yp, yt).reshape(_NW, 2, 16, 6)
    qn = jnp.sum(acc[:, 0], axis=(0, 1))
    sn = jnp.sum(acc[:, 1], axis=(0, 1))
    q = qn[1:5]
    s = sn[1:5]
    mse = q / jnp.maximum(s, 1.0)
    valid = s > 0
    mses = jnp.where(valid, mse, 0.0)
    w = jnp.where(valid, 1.0 / jnp.maximum(s, 1.0), 0.0)
    w = w / jnp.sum(w)
    return jnp.sqrt(jnp.sum(w * mses) + 1e-8)


# speedup vs baseline: 3.4329x; 1.3696x over previous
import functools

import jax
import jax.numpy as jnp
import numpy as np
from jax import lax
from jax.experimental import pallas as pl
from jax.experimental.pallas import tpu as pltpu
from jax.experimental.pallas import tpu_sc as plsc

_BINS = [0.0, float(np.log1p(5.0)), float(np.log1p(25.0)),
         float(np.log1p(50.0)), float(np.log1p(100.0))]

_C = 384
_ROWS = 36864
_NW = 32
_CH = 64
_RSC = 20480
_RPW = _RSC // _NW
_NCHUNK = _RPW // _CH
_VPR = _C // 16

_RB = 128
_RTC = _ROWS - _RSC


def _sc_body(yp_hbm, yt_hbm, out_hbm, ypb0, ypb1, ytb0, ytb1,
             accq, accc, s0, s1, s2, s3):
    w = lax.axis_index("s") * 2 + lax.axis_index("c")
    base = w * _RPW
    ypb = [ypb0, ypb1]
    ytb = [ytb0, ytb1]
    sp = [s0, s1]
    st = [s2, s3]

    lanes6 = lax.iota(jnp.int32, 16) * 6
    zero16 = jnp.zeros((16,), jnp.float32)
    one16 = jnp.ones((16,), jnp.float32)
    for k in range(6):
        accq[pl.ds(k * 16, 16)] = zero16
        accc[pl.ds(k * 16, 16)] = zero16

    def issue(c, slot):
        r0 = base + c * _CH
        pltpu.async_copy(yp_hbm.at[pl.ds(r0, _CH)], ypb[slot], sp[slot])
        pltpu.async_copy(yt_hbm.at[pl.ds(r0, _CH)], ytb[slot], st[slot])

    def wait(c, slot):
        r0 = base + c * _CH
        pltpu.make_async_copy(yp_hbm.at[pl.ds(r0, _CH)], ypb[slot], sp[slot]).wait()
        pltpu.make_async_copy(yt_hbm.at[pl.ds(r0, _CH)], ytb[slot], st[slot]).wait()

    issue(0, 0)
    issue(1, 1)

    def chunk_pair(g, carry):
        for slot in range(2):
            c = g * 2 + slot
            wait(c, slot)

            @plsc.parallel_loop(0, _CH, 1, unroll=2)
            def _rows(r):
                for v in range(_VPR):
                    t = ytb[slot][r, pl.ds(v * 16, 16)]
                    p = ypb[slot][r, pl.ds(v * 16, 16)]
                    d = p - t
                    d2 = d * d
                    idx = jnp.zeros((16,), jnp.int32)
                    ione = jnp.ones((16,), jnp.int32)
                    izero = jnp.zeros((16,), jnp.int32)
                    for e in _BINS:
                        idx = idx + jnp.where(t >= e, ione, izero)
                    fidx = lanes6 + idx
                    plsc.addupdate_scatter(accq, [fidx], d2)
                    plsc.addupdate_scatter(accc, [fidx], one16)

            @pl.when(c + 2 < _NCHUNK)
            def _():
                issue(c + 2, slot)
        return carry

    lax.fori_loop(0, _NCHUNK // 2, chunk_pair, 0)

    pltpu.sync_copy(accq, out_hbm.at[w, 0])
    pltpu.sync_copy(accc, out_hbm.at[w, 1])


def _sc_partials(yp, yt):
    mesh = plsc.VectorSubcoreMesh(core_axis_name="c", subcore_axis_name="s",
                                  num_cores=2, num_subcores=16)
    f = functools.partial(
        pl.kernel,
        mesh=mesh,
        compiler_params=pltpu.CompilerParams(needs_layout_passes=False),
        out_type=jax.ShapeDtypeStruct((_NW, 2, 96), jnp.float32),
        scratch_types=[
            pltpu.VMEM((_CH, _C), jnp.float32),
            pltpu.VMEM((_CH, _C), jnp.float32),
            pltpu.VMEM((_CH, _C), jnp.float32),
            pltpu.VMEM((_CH, _C), jnp.float32),
            pltpu.VMEM((96,), jnp.float32),
            pltpu.VMEM((96,), jnp.float32),
            pltpu.SemaphoreType.DMA,
            pltpu.SemaphoreType.DMA,
            pltpu.SemaphoreType.DMA,
            pltpu.SemaphoreType.DMA,
        ],
    )(_sc_body)
    return f(yp, yt)


def _tc_body(yp_ref, yt_ref, out_ref):
    i = pl.program_id(0)
    accs = [jnp.zeros((8, _C), jnp.float32) for _ in range(8)]
    for j in range(_RB // 8):
        t = yt_ref[pl.ds(j * 8, 8), :]
        p = yp_ref[pl.ds(j * 8, 8), :]
        d = p - t
        d2 = d * d
        ge = [jnp.where(t >= e, 1.0, 0.0) for e in _BINS]
        for b in range(4):
            ind = ge[b] - ge[b + 1]
            accs[b] = accs[b] + ind * d2
            accs[4 + b] = accs[4 + b] + ind

    @pl.when(i == 0)
    def _init():
        out_ref[...] = jnp.zeros_like(out_ref)

    out_ref[...] += jnp.stack(accs, axis=0)


def _tc_partials(yp, yt):
    base = _RSC // _RB
    return pl.pallas_call(
        _tc_body,
        grid=(_RTC // _RB,),
        in_specs=[pl.BlockSpec((_RB, _C), lambda i: (base + i, 0))] * 2,
        out_specs=pl.BlockSpec((8, 8, _C), lambda i: (0, 0, 0)),
        out_shape=jax.ShapeDtypeStruct((8, 8, _C), jnp.float32),
    )(yp, yt)


def kernel(y_pred, y_true):
    yp = y_pred.reshape(_ROWS, _C)
    yt = y_true.reshape(_ROWS, _C)
    sc = _sc_partials(yp, yt).reshape(_NW, 2, 16, 6)
    tc = _tc_partials(yp, yt)
    qn = jnp.sum(sc[:, 0], axis=(0, 1))
    sn = jnp.sum(sc[:, 1], axis=(0, 1))
    q = qn[1:5] + jnp.sum(tc[0:4], axis=(1, 2))
    s = sn[1:5] + jnp.sum(tc[4:8], axis=(1, 2))
    mse = q / jnp.maximum(s, 1.0)
    valid = s > 0
    mses = jnp.where(valid, mse, 0.0)
    w = jnp.where(valid, 1.0 / jnp.maximum(s, 1.0), 0.0)
    w = w / jnp.sum(w)
    return jnp.sqrt(jnp.sum(w * mses) + 1e-8)
